# Initial kernel scaffold; baseline (speedup 1.0000x reference)
import jax
import jax.numpy as jnp
from jax.experimental import pallas as pl


def kernel(users_emb, items_emb, edge_index, graph_vals, diag_vals, W, b):
    raise NotImplementedError("write your pallas kernel here")


# speedup vs baseline: 1.8104x; 1.8104x over previous
import functools

import jax
import jax.numpy as jnp
from jax import lax
from jax.experimental import pallas as pl
from jax.experimental.pallas import tpu as pltpu
from jax.experimental.pallas import tpu_sc as plsc

N_NODES = 10000
N_EDGES = 320000
D = 128
H = 64
NS = 16
EPT = N_EDGES // NS
CH = 10000
NST = EPT // CH
K = 80
NBLK = CH // K
NCHUNK = 624
NTAIL = N_NODES - NCHUNK * NS

_SC_PARAMS = dict(
    compiler_params=pltpu.CompilerParams(needs_layout_passes=False,
                                         use_tc_tiling_on_sc=False),
)


def _i0():
    return jnp.int32(0)


def _mesh():
    return plsc.VectorSubcoreMesh(core_axis_name="c", subcore_axis_name="s")


def _zero_accs(zz_ref, accs, s):
    nb = s * jnp.int32(NCHUNK)
    for acc in accs:
        pltpu.sync_copy(zz_ref.at[pl.ds(jnp.int32(0), NCHUNK)],
                        acc.at[pl.ds(nb, NCHUNK)])

    @pl.when(s == 0)
    def _zero_tail():
        for acc in accs:
            pltpu.sync_copy(zz_ref.at[pl.ds(jnp.int32(0), NTAIL)],
                            acc.at[pl.ds(jnp.int32(NCHUNK * NS), NTAIL)])


def _copy_idx16(dst_ref, src_ref, src_off, n, scale2=None):
    for i in range(n // 16):
        sl = pl.ds(src_off + i * 16, 16)
        v = src_ref[sl]
        if scale2 is not None:
            v = v * 2 + scale2
        dst_ref[pl.ds(i * 16, 16)] = v


def _sc_sum_pow(x_r, x2_r, src_h, dst_h, zz):
    @functools.partial(
        pl.kernel,
        mesh=_mesh(),
        out_type=jax.ShapeDtypeStruct((2, 2, N_NODES, H), jnp.float32),
        scratch_types=[
            pltpu.VMEM((CH,), jnp.int32),
            pltpu.VMEM((CH,), jnp.int32),
            pltpu.VMEM((K,), jnp.int32),
            pltpu.VMEM((K,), jnp.int32),
            pltpu.VMEM((K, H), jnp.float32),
            pltpu.VMEM((K, H), jnp.float32),
            pltpu.VMEM_SHARED((N_NODES, H), jnp.float32),
            pltpu.VMEM_SHARED((N_NODES, H), jnp.float32),
            pltpu.SemaphoreType.DMA,
            pltpu.SemaphoreType.DMA,
        ],
        **_SC_PARAMS,
    )
    def sc_fn(x_ref, x2_ref, src_ref, dst_ref, zz_ref, out_ref,
              srcS, dstS, srcv, dstv, rows, rows2, accs, accp, sem1, sem2):
        c = lax.axis_index("c")
        s = lax.axis_index("s")
        _zero_accs(zz_ref, (accs, accp), s)
        plsc.subcore_barrier()

        ebase = s * jnp.int32(EPT)

        def stage_body(t, carry):
            sb = ebase + t * jnp.int32(CH)
            pltpu.sync_copy(src_ref.at[pl.ds(sb, CH)], srcS)
            pltpu.sync_copy(dst_ref.at[pl.ds(sb, CH)], dstS)

            def blk_body(j, carry2):
                base = j * jnp.int32(K)
                _copy_idx16(srcv, srcS, base, K, scale2=c)
                _copy_idx16(dstv, dstS, base, K)
                cp1 = pltpu.async_copy(x_ref.at[srcv], rows, sem1)
                cp2 = pltpu.async_copy(x2_ref.at[srcv], rows2, sem2)
                cp1.wait()
                cp2.wait()
                pltpu.sync_copy(rows, accs.at[dstv], add=True)
                pltpu.sync_copy(rows2, accp.at[dstv], add=True)
                return carry2

            lax.fori_loop(jnp.int32(0), jnp.int32(NBLK), blk_body, jnp.int32(0))
            return carry

        lax.fori_loop(jnp.int32(0), jnp.int32(NST), stage_body, jnp.int32(0))
        plsc.subcore_barrier()

        nb = s * jnp.int32(NCHUNK)
        for q, acc in enumerate((accs, accp)):
            pltpu.sync_copy(acc.at[pl.ds(nb, NCHUNK)],
                            out_ref.at[c, jnp.int32(q), pl.ds(nb, NCHUNK)])

        @pl.when(s == 0)
        def _write_tail():
            tb = jnp.int32(NCHUNK * NS)
            for q, acc in enumerate((accs, accp)):
                pltpu.sync_copy(acc.at[pl.ds(tb, NTAIL)],
                                out_ref.at[c, jnp.int32(q), pl.ds(tb, NTAIL)])

    return sc_fn(x_r, x2_r, src_h, dst_h, zz)


def _sc_gcn(x_r, src_h, dst_h, val_h, zz):
    @functools.partial(
        pl.kernel,
        mesh=_mesh(),
        out_type=jax.ShapeDtypeStruct((2, N_NODES, H), jnp.float32),
        scratch_types=[
            pltpu.VMEM((CH,), jnp.int32),
            pltpu.VMEM((CH,), jnp.int32),
            pltpu.VMEM((CH,), jnp.float32),
            pltpu.VMEM((K,), jnp.int32),
            pltpu.VMEM((K,), jnp.int32),
            pltpu.VMEM((K, H), jnp.float32),
            pltpu.VMEM((K, H), jnp.float32),
            pltpu.VMEM_SHARED((N_NODES, H), jnp.float32),
            pltpu.SemaphoreType.DMA,
        ],
        **_SC_PARAMS,
    )
    def sc_fn(x_ref, src_ref, dst_ref, val_ref, zz_ref, out_ref,
              srcS, dstS, valS, srcv, dstv, rows, gcnb, accg, sem1):
        c = lax.axis_index("c")
        s = lax.axis_index("s")
        _zero_accs(zz_ref, (accg,), s)
        plsc.subcore_barrier()

        iota16 = lax.iota(jnp.int32, 16)
        ebase = s * jnp.int32(EPT)

        def stage_body(t, carry):
            sb = ebase + t * jnp.int32(CH)
            pltpu.sync_copy(src_ref.at[pl.ds(sb, CH)], srcS)
            pltpu.sync_copy(dst_ref.at[pl.ds(sb, CH)], dstS)
            pltpu.sync_copy(val_ref.at[pl.ds(sb, CH)], valS)

            def blk_body(j, carry2):
                base = j * jnp.int32(K)
                _copy_idx16(srcv, srcS, base, K, scale2=c)
                _copy_idx16(dstv, dstS, base, K)
                pltpu.async_copy(x_ref.at[srcv], rows, sem1).wait()
                v16s = [valS[pl.ds(base + g * 16, 16)] for g in range(K // 16)]

                def col_body(col, carry3):
                    ci = jnp.full((16,), col, jnp.int32)
                    for g in range(K // 16):
                        eidx = iota16 + (g * 16)
                        r = plsc.load_gather(rows, [eidx, ci])
                        plsc.store_scatter(gcnb, [eidx, ci], r * v16s[g])
                    return carry3

                lax.fori_loop(jnp.int32(0), jnp.int32(H), col_body, jnp.int32(0))
                pltpu.sync_copy(gcnb, accg.at[dstv], add=True)
                return carry2

            lax.fori_loop(jnp.int32(0), jnp.int32(NBLK), blk_body, jnp.int32(0))
            return carry

        lax.fori_loop(jnp.int32(0), jnp.int32(NST), stage_body, jnp.int32(0))
        plsc.subcore_barrier()

        nb = s * jnp.int32(NCHUNK)
        pltpu.sync_copy(accg.at[pl.ds(nb, NCHUNK)],
                        out_ref.at[c, pl.ds(nb, NCHUNK)])

        @pl.when(s == 0)
        def _write_tail():
            tb = jnp.int32(NCHUNK * NS)
            pltpu.sync_copy(accg.at[pl.ds(tb, NTAIL)],
                            out_ref.at[c, pl.ds(tb, NTAIL)])

    return sc_fn(x_r, src_h, dst_h, val_h, zz)


def _square_kernel(x):
    bn = 1000

    def body(x_ref, o_ref):
        v = x_ref[...]
        o_ref[...] = v * v

    return pl.pallas_call(
        body,
        grid=(N_NODES // bn,),
        in_specs=[pl.BlockSpec((bn, D), lambda i: (i, _i0()))],
        out_specs=pl.BlockSpec((bn, D), lambda i: (i, _i0())),
        out_shape=jax.ShapeDtypeStruct((N_NODES, D), jnp.float32),
    )(x)


def _epilogue_kernel(g0, g1, s0, s1, p0, p1, diag, Wg0, Wg1, Wp0, Wp1, b):
    bn = 400

    def body(g0_r, g1_r, s0_r, s1_r, p0_r, p1_r, d_r, wg0_r, wg1_r,
             wp0_r, wp1_r, b_r, o_r):
        d = d_r[...]
        pna0 = 0.5 * d * (s0_r[...] * s0_r[...] - p0_r[...])
        pna1 = 0.5 * d * (s1_r[...] * s1_r[...] - p1_r[...])
        h = jnp.dot(g0_r[...], wg0_r[...], preferred_element_type=jnp.float32)
        h += jnp.dot(g1_r[...], wg1_r[...], preferred_element_type=jnp.float32)
        h += jnp.dot(pna0, wp0_r[...], preferred_element_type=jnp.float32)
        h += jnp.dot(pna1, wp1_r[...], preferred_element_type=jnp.float32)
        h += b_r[...]
        o_r[...] = jnp.where(h > 0, h, 0.2 * h)

    half = pl.BlockSpec((bn, H), lambda i: (i, _i0()))
    wspec = pl.BlockSpec((H, D), lambda i: (_i0(), _i0()))
    return pl.pallas_call(
        body,
        grid=(N_NODES // bn,),
        in_specs=[half, half, half, half, half, half,
                  pl.BlockSpec((bn, 1), lambda i: (i, _i0())),
                  wspec, wspec, wspec, wspec,
                  pl.BlockSpec((1, D), lambda i: (_i0(), _i0()))],
        out_specs=pl.BlockSpec((bn, D), lambda i: (i, _i0())),
        out_shape=jax.ShapeDtypeStruct((N_NODES, D), jnp.float32),
    )(g0, g1, s0, s1, p0, p1, diag, Wg0, Wg1, Wp0, Wp1, b)


def kernel(users_emb, items_emb, edge_index, graph_vals, diag_vals, W, b):
    num_user = users_emb.shape[0]
    x = jnp.concatenate([users_emb, items_emb], axis=0)
    x2 = _square_kernel(x)
    x_r = x.reshape(2 * N_NODES, H)
    x2_r = x2.reshape(2 * N_NODES, H)
    dst32 = edge_index[0].astype(jnp.int32)
    src32 = edge_index[1].astype(jnp.int32)
    val32 = graph_vals.astype(jnp.float32)
    zz = jnp.zeros((NCHUNK, H), jnp.float32)

    osp = _sc_sum_pow(x_r, x2_r, src32, dst32, zz)
    og = _sc_gcn(x_r, src32, dst32, val32, zz)

    diag = diag_vals.astype(jnp.float32).reshape(N_NODES, 1)
    Wf = W.astype(jnp.float32)
    Wg0, Wg1 = Wf[:H], Wf[H:D]
    Wp0, Wp1 = Wf[D:D + H], Wf[D + H:]
    b2 = b.astype(jnp.float32).reshape(1, D)

    out = _epilogue_kernel(og[0], og[1], osp[0, 0], osp[1, 0], osp[0, 1],
                           osp[1, 1], diag, Wg0, Wg1, Wp0, Wp1, b2)
    out64 = out.astype(jnp.float64)
    return (out64[:num_user], out64[num_user:])


# speedup vs baseline: 2.1421x; 1.1832x over previous
import functools

import jax
import jax.numpy as jnp
from jax import lax
from jax.experimental import pallas as pl
from jax.experimental.pallas import tpu as pltpu
from jax.experimental.pallas import tpu_sc as plsc

N_NODES = 10000
N_EDGES = 320000
D = 128
H = 64
NS = 16
EPT = N_EDGES // NS
CH = 10000
NST = EPT // CH
K = 80
NBLK = CH // K
NPAIR = (NBLK - 1) // 2
NCHUNK = 624
NTAIL = N_NODES - NCHUNK * NS

_SC_PARAMS = dict(
    compiler_params=pltpu.CompilerParams(needs_layout_passes=False,
                                         use_tc_tiling_on_sc=False),
)


def _i0():
    return jnp.int32(0)


def _mesh():
    return plsc.VectorSubcoreMesh(core_axis_name="c", subcore_axis_name="s")


def _zero_accs(zz_ref, accs, s):
    nb = s * jnp.int32(NCHUNK)
    for acc in accs:
        pltpu.sync_copy(zz_ref.at[pl.ds(_i0(), NCHUNK)],
                        acc.at[pl.ds(nb, NCHUNK)])

    @pl.when(s == 0)
    def _zero_tail():
        for acc in accs:
            pltpu.sync_copy(zz_ref.at[pl.ds(_i0(), NTAIL)],
                            acc.at[pl.ds(jnp.int32(NCHUNK * NS), NTAIL)])


def _write_accs(out_slices, accs, s):
    nb = s * jnp.int32(NCHUNK)
    for out_sl, acc in zip(out_slices, accs):
        pltpu.sync_copy(acc.at[pl.ds(nb, NCHUNK)], out_sl(nb, NCHUNK))

    @pl.when(s == 0)
    def _write_tail():
        tb = jnp.int32(NCHUNK * NS)
        for out_sl, acc in zip(out_slices, accs):
            pltpu.sync_copy(acc.at[pl.ds(tb, NTAIL)], out_sl(tb, NTAIL))


def _copy_idx16(dst_ref, src_ref, src_off, scale2=None):
    for i in range(K // 16):
        sl = pl.ds(src_off + i * 16, 16)
        v = src_ref[sl]
        if scale2 is not None:
            v = v * 2 + scale2
        dst_ref[pl.ds(i * 16, 16)] = v


def _square_rows(dst_ref, src_ref):
    def body(e2, carry):
        e = e2 * jnp.int32(2)
        for r in range(2):
            for ci in range(H // 16):
                sl = pl.ds(jnp.int32(ci * 16), 16)
                v = src_ref[e + r, sl]
                dst_ref[e + r, sl] = v * v
        return carry

    lax.fori_loop(_i0(), jnp.int32(K // 2), body, _i0())


def _sc_sum_pow(x_r, src_h, dst_h, zz):
    @functools.partial(
        pl.kernel,
        mesh=_mesh(),
        out_type=jax.ShapeDtypeStruct((2, 2, N_NODES, H), jnp.float32),
        scratch_types=[
            pltpu.VMEM((CH,), jnp.int32),
            pltpu.VMEM((CH,), jnp.int32),
            pltpu.VMEM((K,), jnp.int32),
            pltpu.VMEM((K,), jnp.int32),
            pltpu.VMEM((K,), jnp.int32),
            pltpu.VMEM((K,), jnp.int32),
            pltpu.VMEM((K, H), jnp.float32),
            pltpu.VMEM((K, H), jnp.float32),
            pltpu.VMEM((K, H), jnp.float32),
            pltpu.VMEM_SHARED((N_NODES, H), jnp.float32),
            pltpu.VMEM_SHARED((N_NODES, H), jnp.float32),
            pltpu.SemaphoreType.DMA,
            pltpu.SemaphoreType.DMA,
        ],
        **_SC_PARAMS,
    )
    def sc_fn(x_ref, src_ref, dst_ref, zz_ref, out_ref,
              srcS, dstS, srcv0, srcv1, dstv0, dstv1, rows0, rows1, sq,
              accs, accp, semg0, semg1):
        c = lax.axis_index("c")
        s = lax.axis_index("s")
        _zero_accs(zz_ref, (accs, accp), s)
        plsc.subcore_barrier()

        srcv = (srcv0, srcv1)
        dstv = (dstv0, dstv1)
        rows = (rows0, rows1)
        semg = (semg0, semg1)
        ebase = s * jnp.int32(EPT)

        def process(p):
            pltpu.make_async_copy(x_ref.at[srcv[p]], rows[p], semg[p]).wait()
            _square_rows(sq, rows[p])
            pltpu.sync_copy(rows[p], accs.at[dstv[p]], add=True)
            pltpu.sync_copy(sq, accp.at[dstv[p]], add=True)

        def prefetch(q, base):
            _copy_idx16(srcv[q], srcS, base, scale2=c)
            _copy_idx16(dstv[q], dstS, base)
            pltpu.async_copy(x_ref.at[srcv[q]], rows[q], semg[q])

        for t in range(NST):
            sb = ebase + jnp.int32(t * CH)
            pltpu.sync_copy(src_ref.at[pl.ds(sb, CH)], srcS)
            pltpu.sync_copy(dst_ref.at[pl.ds(sb, CH)], dstS)
            prefetch(0, _i0())

            def pair_body(jp, carry):
                j2 = jp * jnp.int32(2 * K)
                for p in range(2):
                    prefetch(1 - p, j2 + jnp.int32((p + 1) * K))
                    process(p)
                return carry

            lax.fori_loop(_i0(), jnp.int32(NPAIR), pair_body, _i0())
            process(0)

        plsc.subcore_barrier()
        _write_accs(
            (lambda nb, nn: out_ref.at[c, _i0(), pl.ds(nb, nn)],
             lambda nb, nn: out_ref.at[c, jnp.int32(1), pl.ds(nb, nn)]),
            (accs, accp), s)

    return sc_fn(x_r, src_h, dst_h, zz)


def _sc_gcn(x_r, src_h, dst_h, val_h, zz):
    @functools.partial(
        pl.kernel,
        mesh=_mesh(),
        out_type=jax.ShapeDtypeStruct((2, N_NODES, H), jnp.float32),
        scratch_types=[
            pltpu.VMEM((CH,), jnp.int32),
            pltpu.VMEM((CH,), jnp.int32),
            pltpu.VMEM((CH,), jnp.float32),
            pltpu.VMEM((K,), jnp.int32),
            pltpu.VMEM((K,), jnp.int32),
            pltpu.VMEM((K,), jnp.int32),
            pltpu.VMEM((K,), jnp.int32),
            pltpu.VMEM((K, H), jnp.float32),
            pltpu.VMEM((K, H), jnp.float32),
            pltpu.VMEM((K, H), jnp.float32),
            pltpu.VMEM_SHARED((N_NODES, H), jnp.float32),
            pltpu.SemaphoreType.DMA,
            pltpu.SemaphoreType.DMA,
        ],
        **_SC_PARAMS,
    )
    def sc_fn(x_ref, src_ref, dst_ref, val_ref, zz_ref, out_ref,
              srcS, dstS, valS, srcv0, srcv1, dstv0, dstv1,
              rows0, rows1, gcnb, accg, semg0, semg1):
        c = lax.axis_index("c")
        s = lax.axis_index("s")
        _zero_accs(zz_ref, (accg,), s)
        plsc.subcore_barrier()

        srcv = (srcv0, srcv1)
        dstv = (dstv0, dstv1)
        rows = (rows0, rows1)
        semg = (semg0, semg1)
        iota16 = lax.iota(jnp.int32, 16)
        ebase = s * jnp.int32(EPT)

        def process(p, base):
            pltpu.make_async_copy(x_ref.at[srcv[p]], rows[p], semg[p]).wait()
            v16s = [valS[pl.ds(base + g * 16, 16)] for g in range(K // 16)]

            def col_body(col, carry3):
                ci = jnp.full((16,), col, jnp.int32)
                for g in range(K // 16):
                    eidx = iota16 + (g * 16)
                    r = plsc.load_gather(rows[p], [eidx, ci])
                    plsc.store_scatter(gcnb, [eidx, ci], r * v16s[g])
                return carry3

            lax.fori_loop(_i0(), jnp.int32(H), col_body, _i0())
            pltpu.sync_copy(gcnb, accg.at[dstv[p]], add=True)

        def prefetch(q, base):
            _copy_idx16(srcv[q], srcS, base, scale2=c)
            _copy_idx16(dstv[q], dstS, base)
            pltpu.async_copy(x_ref.at[srcv[q]], rows[q], semg[q])

        for t in range(NST):
            sb = ebase + jnp.int32(t * CH)
            pltpu.sync_copy(src_ref.at[pl.ds(sb, CH)], srcS)
            pltpu.sync_copy(dst_ref.at[pl.ds(sb, CH)], dstS)
            pltpu.sync_copy(val_ref.at[pl.ds(sb, CH)], valS)
            prefetch(0, _i0())

            def pair_body(jp, carry):
                j2 = jp * jnp.int32(2 * K)
                for p in range(2):
                    prefetch(1 - p, j2 + jnp.int32((p + 1) * K))
                    process(p, j2 + jnp.int32(p * K))
                return carry

            lax.fori_loop(_i0(), jnp.int32(NPAIR), pair_body, _i0())
            process(0, jnp.int32((NBLK - 1) * K))

        plsc.subcore_barrier()
        _write_accs((lambda nb, nn: out_ref.at[c, pl.ds(nb, nn)],),
                    (accg,), s)

    return sc_fn(x_r, src_h, dst_h, val_h, zz)


def _epilogue_kernel(g0, g1, s0, s1, p0, p1, diag, Wg0, Wg1, Wp0, Wp1, b):
    bn = 400

    def body(g0_r, g1_r, s0_r, s1_r, p0_r, p1_r, d_r, wg0_r, wg1_r,
             wp0_r, wp1_r, b_r, o_r):
        d = d_r[...]
        pna0 = 0.5 * d * (s0_r[...] * s0_r[...] - p0_r[...])
        pna1 = 0.5 * d * (s1_r[...] * s1_r[...] - p1_r[...])
        h = jnp.dot(g0_r[...], wg0_r[...], preferred_element_type=jnp.float32)
        h += jnp.dot(g1_r[...], wg1_r[...], preferred_element_type=jnp.float32)
        h += jnp.dot(pna0, wp0_r[...], preferred_element_type=jnp.float32)
        h += jnp.dot(pna1, wp1_r[...], preferred_element_type=jnp.float32)
        h += b_r[...]
        o_r[...] = jnp.where(h > 0, h, 0.2 * h)

    half = pl.BlockSpec((bn, H), lambda i: (i, _i0()))
    wspec = pl.BlockSpec((H, D), lambda i: (_i0(), _i0()))
    return pl.pallas_call(
        body,
        grid=(N_NODES // bn,),
        in_specs=[half, half, half, half, half, half,
                  pl.BlockSpec((bn, 1), lambda i: (i, _i0())),
                  wspec, wspec, wspec, wspec,
                  pl.BlockSpec((1, D), lambda i: (_i0(), _i0()))],
        out_specs=pl.BlockSpec((bn, D), lambda i: (i, _i0())),
        out_shape=jax.ShapeDtypeStruct((N_NODES, D), jnp.float32),
    )(g0, g1, s0, s1, p0, p1, diag, Wg0, Wg1, Wp0, Wp1, b)


def kernel(users_emb, items_emb, edge_index, graph_vals, diag_vals, W, b):
    num_user = users_emb.shape[0]
    x = jnp.concatenate([users_emb, items_emb], axis=0)
    x_r = x.reshape(2 * N_NODES, H)
    dst32 = edge_index[0].astype(jnp.int32)
    src32 = edge_index[1].astype(jnp.int32)
    val32 = graph_vals.astype(jnp.float32)
    zz = jnp.zeros((NCHUNK, H), jnp.float32)

    osp = _sc_sum_pow(x_r, src32, dst32, zz)
    og = _sc_gcn(x_r, src32, dst32, val32, zz)

    diag = diag_vals.astype(jnp.float32).reshape(N_NODES, 1)
    Wf = W.astype(jnp.float32)
    Wg0, Wg1 = Wf[:H], Wf[H:D]
    Wp0, Wp1 = Wf[D:D + H], Wf[D + H:]
    b2 = b.astype(jnp.float32).reshape(1, D)

    out = _epilogue_kernel(og[0], og[1], osp[0, 0], osp[1, 0], osp[0, 1],
                           osp[1, 1], diag, Wg0, Wg1, Wp0, Wp1, b2)
    out64 = out.astype(jnp.float64)
    return (out64[:num_user], out64[num_user:])
